# baseline (speedup 1.0000x reference)
import functools

import jax
import jax.numpy as jnp
from jax import lax
from jax.experimental import pallas as pl
from jax.experimental.pallas import tpu as pltpu
from jax.experimental.pallas import tpu_sc as plsc

N = 10000
E = 320000
IN_DIM = 128
OUT_DIM = 32
HEADS = 4
NUM_REL = 48
EPS = 1e-16
HD = HEADS * OUT_DIM
SW = 16
CW = HD + SW
NC = 2
NS = 16
NW = NC * NS
EPW = E // NW
CH = 80
NCHUNK = EPW // CH
RPT = 624
TAIL = N - NS * RPT


def _k1_body(x_ref, wcat_ref, a2_ref, brow_ref, p_ref, s3_ref):
    x = x_ref[...]
    p = lax.dot_general(x, wcat_ref[...], (((1,), (1,)), ((), ())),
                        preferred_element_type=jnp.float32)
    p_ref[...] = p
    s3 = lax.dot_general(p, a2_ref[...], (((1,), (1,)), ((), ())),
                         preferred_element_type=jnp.float32)
    s3_ref[...] = s3 + brow_ref[...]


def _k1(x, wcat, a2, brow):
    blk = 1000
    grid = N // blk
    return pl.pallas_call(
        _k1_body,
        grid=(grid,),
        in_specs=[
            pl.BlockSpec((blk, IN_DIM), lambda i: (i, 0)),
            pl.BlockSpec((HD, IN_DIM), lambda i: (0, 0)),
            pl.BlockSpec((NUM_REL * SW, HD), lambda i: (0, 0)),
            pl.BlockSpec((1, NUM_REL * SW), lambda i: (0, 0)),
        ],
        out_specs=[
            pl.BlockSpec((blk, HD), lambda i: (i, 0)),
            pl.BlockSpec((blk, NUM_REL * SW), lambda i: (i, 0)),
        ],
        out_shape=[
            jax.ShapeDtypeStruct((N, HD), jnp.float32),
            jax.ShapeDtypeStruct((N, NUM_REL * SW), jnp.float32),
        ],
    )(x, wcat, a2, brow)


def _k2_body(s3_hbm, p_hbm, src_hbm, dst_hbm, rt_hbm,
             accs_hbm,
             comb_sh,
             src_v, dst_v, rt_v, sidx_v, dsti_v, srow_v, prow_v, msgc_v,
             esem, gsem, msem):
    cid = lax.axis_index("c")
    sid = lax.axis_index("s")
    wid = sid * NC + cid

    def _zrow(i, _):
        for k in range(CW // 16):
            msgc_v[i, pl.ds(16 * k, 16)] = jnp.zeros((16,), jnp.float32)
        return 0
    lax.fori_loop(0, CH, _zrow, 0)

    for j in range(7):
        pltpu.sync_copy(msgc_v, comb_sh.at[pl.ds(sid * RPT + j * CH, CH)])
    pltpu.sync_copy(msgc_v.at[pl.ds(0, 64)], comb_sh.at[pl.ds(sid * RPT + 7 * CH, 64)])

    @pl.when(sid == 0)
    def _ztail():
        pltpu.sync_copy(msgc_v.at[pl.ds(0, TAIL)], comb_sh.at[pl.ds(NS * RPT, TAIL)])

    plsc.subcore_barrier()

    lanes = lax.iota(jnp.int32, 16)
    expmask = lanes < HEADS

    def fire_l(c, b):
        base = wid * EPW + c * CH
        pltpu.async_copy(src_hbm.at[pl.ds(base, CH)], src_v[b], esem[b])
        pltpu.async_copy(dst_hbm.at[pl.ds(base, CH)], dst_v[b], esem[b])
        pltpu.async_copy(rt_hbm.at[pl.ds(base, CH)], rt_v[b], esem[b])

    def wait_l(b):
        pltpu.make_async_copy(src_hbm.at[pl.ds(0, CH)], src_v[b], esem[b]).wait()
        pltpu.make_async_copy(dst_hbm.at[pl.ds(0, CH)], dst_v[b], esem[b]).wait()
        pltpu.make_async_copy(rt_hbm.at[pl.ds(0, CH)], rt_v[b], esem[b]).wait()

    def do_x(b):
        for i in range(CH // 16):
            sl = pl.ds(16 * i, 16)
            sidx_v[b][sl] = src_v[b][sl] * NUM_REL + rt_v[b][sl]
            dsti_v[b][sl] = dst_v[b][sl]

    def fire_g(b):
        pltpu.async_copy(s3_hbm.at[sidx_v[b]], srow_v[b], gsem[b])
        pltpu.async_copy(p_hbm.at[src_v[b]], prow_v[b], gsem[b])

    def wait_g(b):
        pltpu.make_async_copy(s3_hbm.at[sidx_v[b]], srow_v[b], gsem[b]).wait()
        pltpu.make_async_copy(p_hbm.at[src_v[b]], prow_v[b], gsem[b]).wait()

    def do_c(b):
        def _exprow(e, _):
            row = srow_v[b][e, pl.ds(0, 16)]
            tt = jnp.exp(jnp.maximum(row, 0.2 * row))
            srow_v[b][e, pl.ds(0, 16)] = jnp.where(expmask, tt, row)
            return 0
        lax.fori_loop(0, CH, _exprow, 0)

        def _scale(q, _):
            for u in range(2):
                e = q * 2 + u
                row = srow_v[b][e, pl.ds(0, 16)]
                for h in range(HEADS):
                    exs = row[h]
                    s0 = pl.ds(h * 2 * 16, 16)
                    s1 = pl.ds((h * 2 + 1) * 16, 16)
                    msgc_v[e, s0] = prow_v[b][e, s0] * exs
                    msgc_v[e, s1] = prow_v[b][e, s1] * exs
                msgc_v[e, pl.ds(HD, 16)] = row
            return 0
        lax.fori_loop(0, CH // 2, _scale, 0)

    def fire_s(b):
        pltpu.async_copy(msgc_v, comb_sh.at[dsti_v[b]], msem, add=True)

    def wait_s(b):
        pltpu.make_async_copy(msgc_v, comb_sh.at[dsti_v[b]], msem).wait()

    def sub_a(b):
        wait_l(b)
        do_x(b)
        fire_g(b)

    def sub_b(c, b, first=False, last=False):
        wait_g(b)
        if not first:
            wait_s(1 - b)
        do_c(b)
        fire_s(b)
        if not last:
            if isinstance(c, int):
                if c + 2 < NCHUNK:
                    fire_l(c + 2, b)
            else:
                @pl.when(c + 2 < NCHUNK)
                def _():
                    fire_l(c + 2, b)

    fire_l(0, 0)
    fire_l(1, 1)
    sub_a(0)
    sub_b(0, 0, first=True)
    sub_a(1)
    sub_b(1, 1)

    def _pipe(k, _):
        c = 2 * k + 2
        sub_a(0)
        sub_b(c, 0)
        sub_a(1)
        sub_b(c + 1, 1)
        return 0

    lax.fori_loop(0, (NCHUNK - 3) // 2, _pipe, 0)

    sub_a(0)
    sub_b(NCHUNK - 1, 0, last=True)
    wait_s(0)

    plsc.subcore_barrier()

    r0 = sid * RPT
    pltpu.sync_copy(comb_sh.at[pl.ds(r0, RPT)], accs_hbm.at[cid, pl.ds(r0, RPT)])

    @pl.when(sid == 0)
    def _wtail():
        pltpu.sync_copy(comb_sh.at[pl.ds(NS * RPT, TAIL)],
                        accs_hbm.at[cid, pl.ds(NS * RPT, TAIL)])


def _k2(s3r, p, src, dst, rt):
    mesh = plsc.VectorSubcoreMesh(core_axis_name="c", subcore_axis_name="s")
    ivec = pltpu.VMEM((CH,), jnp.int32)
    f = functools.partial(
        pl.kernel,
        out_type=jax.ShapeDtypeStruct((NC, N, CW), jnp.float32),
        mesh=mesh,
        compiler_params=pltpu.CompilerParams(needs_layout_passes=False,
                                             use_tc_tiling_on_sc=False),
        scratch_types=[
            pltpu.VMEM_SHARED((N, CW), jnp.float32),
            [ivec, ivec], [ivec, ivec], [ivec, ivec], [ivec, ivec],
            [ivec, ivec],
            [pltpu.VMEM((CH, SW), jnp.float32), pltpu.VMEM((CH, SW), jnp.float32)],
            [pltpu.VMEM((CH, HD), jnp.float32), pltpu.VMEM((CH, HD), jnp.float32)],
            pltpu.VMEM((CH, CW), jnp.float32),
            [pltpu.SemaphoreType.DMA, pltpu.SemaphoreType.DMA],
            [pltpu.SemaphoreType.DMA, pltpu.SemaphoreType.DMA],
            pltpu.SemaphoreType.DMA,
        ],
    )(_k2_body)
    return f(s3r, p, src, dst, rt)


def _k3_body(acc_ref, d_ref, out_ref):
    a = acc_ref[0] + acc_ref[1]
    d = d_ref[0] + d_ref[1]
    bias = d[:, 4:5]
    parts = []
    for h in range(HEADS):
        den = d[:, h:h + 1] + EPS
        parts.append(a[:, h * OUT_DIM:(h + 1) * OUT_DIM] / den + bias)
    out_ref[...] = jnp.concatenate(parts, axis=1)


def _k3(acc, dacc):
    blk = 1000
    grid = N // blk
    return pl.pallas_call(
        _k3_body,
        grid=(grid,),
        in_specs=[
            pl.BlockSpec((NC, blk, HD), lambda i: (0, i, 0)),
            pl.BlockSpec((NC, blk, SW), lambda i: (0, i, 0)),
        ],
        out_specs=pl.BlockSpec((blk, HD), lambda i: (i, 0)),
        out_shape=jax.ShapeDtypeStruct((N, HD), jnp.float32),
    )(acc, dacc)


def kernel(node_emb, edge_index, edge_type, W, attn_vec, rel_bias):
    wcat = W.reshape(HD, IN_DIM)
    tmp = attn_vec.transpose(1, 0, 2)
    a2 = jnp.zeros((NUM_REL, SW, HEADS, OUT_DIM), jnp.float32)
    a2 = a2.at[:, jnp.arange(HEADS), jnp.arange(HEADS), :].set(tmp)
    a2 = a2.reshape(NUM_REL * SW, HD)
    brow = jnp.zeros((NUM_REL, SW), jnp.float32).at[:, 4].set(rel_bias)
    brow = brow.reshape(1, NUM_REL * SW)

    p, s3 = _k1(node_emb, wcat, a2, brow)
    s3r = s3.reshape(N * NUM_REL, SW)

    src = edge_index[0].astype(jnp.int32)
    dst = edge_index[1].astype(jnp.int32)
    rt = edge_type.astype(jnp.int32)

    combs = _k2(s3r, p, src, dst, rt)
    return _k3(combs[:, :, :HD], combs[:, :, HD:])


# speedup vs baseline: 1.0562x; 1.0562x over previous
import functools

import jax
import jax.numpy as jnp
from jax import lax
from jax.experimental import pallas as pl
from jax.experimental.pallas import tpu as pltpu
from jax.experimental.pallas import tpu_sc as plsc

N = 10000
E = 320000
IN_DIM = 128
OUT_DIM = 32
HEADS = 4
NUM_REL = 48
EPS = 1e-16
HD = HEADS * OUT_DIM
SW = 16
CW = HD + SW
NC = 2
NS = 16
NW = NC * NS
EPW = E // NW
CH = 80
NCHUNK = EPW // CH
RPT = 624
TAIL = N - NS * RPT


def _k1_body(x_ref, wcat_ref, a2_ref, brow_ref, p_ref, s3_ref):
    x = x_ref[...]
    p = lax.dot_general(x, wcat_ref[...], (((1,), (1,)), ((), ())),
                        preferred_element_type=jnp.float32)
    p_ref[...] = p
    s3 = lax.dot_general(p, a2_ref[...], (((1,), (1,)), ((), ())),
                         preferred_element_type=jnp.float32)
    s3_ref[...] = s3 + brow_ref[...]


def _k1(x, wcat, a2, brow):
    blk = 1000
    grid = N // blk
    return pl.pallas_call(
        _k1_body,
        grid=(grid,),
        in_specs=[
            pl.BlockSpec((blk, IN_DIM), lambda i: (i, 0)),
            pl.BlockSpec((HD, IN_DIM), lambda i: (0, 0)),
            pl.BlockSpec((NUM_REL * SW, HD), lambda i: (0, 0)),
            pl.BlockSpec((1, NUM_REL * SW), lambda i: (0, 0)),
        ],
        out_specs=[
            pl.BlockSpec((blk, HD), lambda i: (i, 0)),
            pl.BlockSpec((blk, NUM_REL * SW), lambda i: (i, 0)),
        ],
        out_shape=[
            jax.ShapeDtypeStruct((N, HD), jnp.float32),
            jax.ShapeDtypeStruct((N, NUM_REL * SW), jnp.float32),
        ],
    )(x, wcat, a2, brow)


def _k2_body(s3_hbm, p_hbm, src_hbm, dst_hbm, rt_hbm,
             accs_hbm,
             comb_sh,
             src_v, dst_v, rt_v, sidx_v, dsti_v, srow_v, prow_v, msgc_v,
             esem, gsem, msem):
    cid = lax.axis_index("c")
    sid = lax.axis_index("s")
    wid = sid * NC + cid

    def _zrow(i, _):
        for k in range(CW // 16):
            msgc_v[i, pl.ds(16 * k, 16)] = jnp.zeros((16,), jnp.float32)
        return 0
    lax.fori_loop(0, CH, _zrow, 0)

    for j in range(7):
        pltpu.sync_copy(msgc_v, comb_sh.at[pl.ds(sid * RPT + j * CH, CH)])
    pltpu.sync_copy(msgc_v.at[pl.ds(0, 64)], comb_sh.at[pl.ds(sid * RPT + 7 * CH, 64)])

    @pl.when(sid == 0)
    def _ztail():
        pltpu.sync_copy(msgc_v.at[pl.ds(0, TAIL)], comb_sh.at[pl.ds(NS * RPT, TAIL)])

    plsc.subcore_barrier()

    lanes = lax.iota(jnp.int32, 16)
    expmask = lanes < HEADS

    def fire_l(c, b):
        base = wid * EPW + c * CH
        pltpu.async_copy(src_hbm.at[pl.ds(base, CH)], src_v[b], esem[b])
        pltpu.async_copy(dst_hbm.at[pl.ds(base, CH)], dst_v[b], esem[b])
        pltpu.async_copy(rt_hbm.at[pl.ds(base, CH)], rt_v[b], esem[b])

    def wait_l(b):
        pltpu.make_async_copy(src_hbm.at[pl.ds(0, CH)], src_v[b], esem[b]).wait()
        pltpu.make_async_copy(dst_hbm.at[pl.ds(0, CH)], dst_v[b], esem[b]).wait()
        pltpu.make_async_copy(rt_hbm.at[pl.ds(0, CH)], rt_v[b], esem[b]).wait()

    def do_x(b):
        for i in range(CH // 16):
            sl = pl.ds(16 * i, 16)
            sidx_v[b][sl] = src_v[b][sl] * NUM_REL + rt_v[b][sl]
            dsti_v[b][sl] = dst_v[b][sl]

    def fire_g(b):
        pltpu.async_copy(s3_hbm.at[sidx_v[b]], srow_v[b], gsem[b])
        pltpu.async_copy(p_hbm.at[src_v[b]], prow_v[b], gsem[b])

    def wait_g(b):
        pltpu.make_async_copy(s3_hbm.at[sidx_v[b]], srow_v[b], gsem[b]).wait()
        pltpu.make_async_copy(p_hbm.at[src_v[b]], prow_v[b], gsem[b]).wait()

    def do_c(b):
        def _scale(q, _):
            for u in range(4):
                e = q * 4 + u
                row = srow_v[b][e, pl.ds(0, 16)]
                tt = jnp.exp(jnp.maximum(row, 0.2 * row))
                rowx = jnp.where(expmask, tt, row)
                for h in range(HEADS):
                    exs = rowx[h]
                    s0 = pl.ds(h * 2 * 16, 16)
                    s1 = pl.ds((h * 2 + 1) * 16, 16)
                    msgc_v[e, s0] = prow_v[b][e, s0] * exs
                    msgc_v[e, s1] = prow_v[b][e, s1] * exs
                msgc_v[e, pl.ds(HD, 16)] = rowx
            return 0
        lax.fori_loop(0, CH // 4, _scale, 0)

    def fire_s(b):
        pltpu.async_copy(msgc_v, comb_sh.at[dsti_v[b]], msem, add=True)

    def wait_s(b):
        pltpu.make_async_copy(msgc_v, comb_sh.at[dsti_v[b]], msem).wait()

    def sub_a(b):
        wait_l(b)
        do_x(b)
        fire_g(b)

    def sub_b(c, b, first=False, last=False):
        wait_g(b)
        if not first:
            wait_s(1 - b)
        do_c(b)
        fire_s(b)
        if not last:
            if isinstance(c, int):
                if c + 2 < NCHUNK:
                    fire_l(c + 2, b)
            else:
                @pl.when(c + 2 < NCHUNK)
                def _():
                    fire_l(c + 2, b)

    fire_l(0, 0)
    fire_l(1, 1)
    sub_a(0)
    sub_b(0, 0, first=True)
    sub_a(1)
    sub_b(1, 1)

    def _pipe(k, _):
        c = 2 * k + 2
        sub_a(0)
        sub_b(c, 0)
        sub_a(1)
        sub_b(c + 1, 1)
        return 0

    lax.fori_loop(0, (NCHUNK - 3) // 2, _pipe, 0)

    sub_a(0)
    sub_b(NCHUNK - 1, 0, last=True)
    wait_s(0)

    plsc.subcore_barrier()

    r0 = sid * RPT
    pltpu.sync_copy(comb_sh.at[pl.ds(r0, RPT)], accs_hbm.at[cid, pl.ds(r0, RPT)])

    @pl.when(sid == 0)
    def _wtail():
        pltpu.sync_copy(comb_sh.at[pl.ds(NS * RPT, TAIL)],
                        accs_hbm.at[cid, pl.ds(NS * RPT, TAIL)])


def _k2(s3r, p, src, dst, rt):
    mesh = plsc.VectorSubcoreMesh(core_axis_name="c", subcore_axis_name="s")
    ivec = pltpu.VMEM((CH,), jnp.int32)
    f = functools.partial(
        pl.kernel,
        out_type=jax.ShapeDtypeStruct((NC, N, CW), jnp.float32),
        mesh=mesh,
        compiler_params=pltpu.CompilerParams(needs_layout_passes=False,
                                             use_tc_tiling_on_sc=False),
        scratch_types=[
            pltpu.VMEM_SHARED((N, CW), jnp.float32),
            [ivec, ivec], [ivec, ivec], [ivec, ivec], [ivec, ivec],
            [ivec, ivec],
            [pltpu.VMEM((CH, SW), jnp.float32), pltpu.VMEM((CH, SW), jnp.float32)],
            [pltpu.VMEM((CH, HD), jnp.float32), pltpu.VMEM((CH, HD), jnp.float32)],
            pltpu.VMEM((CH, CW), jnp.float32),
            [pltpu.SemaphoreType.DMA, pltpu.SemaphoreType.DMA],
            [pltpu.SemaphoreType.DMA, pltpu.SemaphoreType.DMA],
            pltpu.SemaphoreType.DMA,
        ],
    )(_k2_body)
    return f(s3r, p, src, dst, rt)


def _k3_body(acc_ref, d_ref, out_ref):
    a = acc_ref[0] + acc_ref[1]
    d = d_ref[0] + d_ref[1]
    bias = d[:, 4:5]
    parts = []
    for h in range(HEADS):
        den = d[:, h:h + 1] + EPS
        parts.append(a[:, h * OUT_DIM:(h + 1) * OUT_DIM] / den + bias)
    out_ref[...] = jnp.concatenate(parts, axis=1)


def _k3(acc, dacc):
    blk = 1000
    grid = N // blk
    return pl.pallas_call(
        _k3_body,
        grid=(grid,),
        in_specs=[
            pl.BlockSpec((NC, blk, HD), lambda i: (0, i, 0)),
            pl.BlockSpec((NC, blk, SW), lambda i: (0, i, 0)),
        ],
        out_specs=pl.BlockSpec((blk, HD), lambda i: (i, 0)),
        out_shape=jax.ShapeDtypeStruct((N, HD), jnp.float32),
    )(acc, dacc)


def kernel(node_emb, edge_index, edge_type, W, attn_vec, rel_bias):
    wcat = W.reshape(HD, IN_DIM)
    tmp = attn_vec.transpose(1, 0, 2)
    a2 = jnp.zeros((NUM_REL, SW, HEADS, OUT_DIM), jnp.float32)
    a2 = a2.at[:, jnp.arange(HEADS), jnp.arange(HEADS), :].set(tmp)
    a2 = a2.reshape(NUM_REL * SW, HD)
    brow = jnp.zeros((NUM_REL, SW), jnp.float32).at[:, 4].set(rel_bias)
    brow = brow.reshape(1, NUM_REL * SW)

    p, s3 = _k1(node_emb, wcat, a2, brow)
    s3r = s3.reshape(N * NUM_REL, SW)

    src = edge_index[0].astype(jnp.int32)
    dst = edge_index[1].astype(jnp.int32)
    rt = edge_type.astype(jnp.int32)

    combs = _k2(s3r, p, src, dst, rt)
    return _k3(combs[:, :, :HD], combs[:, :, HD:])


# speedup vs baseline: 2.2474x; 2.1279x over previous
import functools

import jax
import jax.numpy as jnp
from jax import lax
from jax.experimental import pallas as pl
from jax.experimental.pallas import tpu as pltpu
from jax.experimental.pallas import tpu_sc as plsc

N = 10000
E = 320000
IN_DIM = 128
OUT_DIM = 32
HEADS = 4
NUM_REL = 48
EPS = 1e-16
HD = HEADS * OUT_DIM
SW = 16
CW = HD + SW
NC = 2
NS = 16
NW = NC * NS
EPW = E // NW
CH = 80
NCHUNK = EPW // CH
RPT = 624
TAIL = N - NS * RPT


def _k1_body(x_ref, wcat_ref, a2_ref, brow_ref, p_ref, s3_ref):
    x = x_ref[...]
    p = lax.dot_general(x, wcat_ref[...], (((1,), (1,)), ((), ())),
                        preferred_element_type=jnp.float32)
    p_ref[...] = p
    s3 = lax.dot_general(p, a2_ref[...], (((1,), (1,)), ((), ())),
                         preferred_element_type=jnp.float32)
    s3_ref[...] = s3 + brow_ref[...]


def _k1(x, wcat, a2, brow):
    blk = 1000
    grid = N // blk
    return pl.pallas_call(
        _k1_body,
        grid=(grid,),
        in_specs=[
            pl.BlockSpec((blk, IN_DIM), lambda i: (i, 0)),
            pl.BlockSpec((HD, IN_DIM), lambda i: (0, 0)),
            pl.BlockSpec((NUM_REL * SW, HD), lambda i: (0, 0)),
            pl.BlockSpec((1, NUM_REL * SW), lambda i: (0, 0)),
        ],
        out_specs=[
            pl.BlockSpec((blk, HD), lambda i: (i, 0)),
            pl.BlockSpec((blk, NUM_REL * SW), lambda i: (i, 0)),
        ],
        out_shape=[
            jax.ShapeDtypeStruct((N, HD), jnp.float32),
            jax.ShapeDtypeStruct((N, NUM_REL * SW), jnp.float32),
        ],
    )(x, wcat, a2, brow)


def _k2_body(s3_hbm, p_hbm, src_hbm, dst_hbm, rt_hbm,
             accs_hbm,
             comb_sh,
             src_v, dst_v, rt_v, sidx_v, dsti_v, srow_v, prow_v, msgc_v,
             esem, gsem, msem):
    cid = lax.axis_index("c")
    sid = lax.axis_index("s")
    wid = sid * NC + cid

    def _zrow(i, _):
        for k in range(CW // 16):
            msgc_v[i, pl.ds(16 * k, 16)] = jnp.zeros((16,), jnp.float32)
        return 0
    lax.fori_loop(0, CH, _zrow, 0)

    for j in range(7):
        pltpu.sync_copy(msgc_v, comb_sh.at[pl.ds(sid * RPT + j * CH, CH)])
    pltpu.sync_copy(msgc_v.at[pl.ds(0, 64)], comb_sh.at[pl.ds(sid * RPT + 7 * CH, 64)])

    @pl.when(sid == 0)
    def _ztail():
        pltpu.sync_copy(msgc_v.at[pl.ds(0, TAIL)], comb_sh.at[pl.ds(NS * RPT, TAIL)])

    plsc.subcore_barrier()

    lanes = lax.iota(jnp.int32, 16)
    expmask = lanes < HEADS

    def fire_l(c, b):
        base = wid * EPW + c * CH
        pltpu.async_copy(src_hbm.at[pl.ds(base, CH)], src_v[b], esem[b])
        pltpu.async_copy(dst_hbm.at[pl.ds(base, CH)], dst_v[b], esem[b])
        pltpu.async_copy(rt_hbm.at[pl.ds(base, CH)], rt_v[b], esem[b])

    def wait_l(b):
        pltpu.make_async_copy(src_hbm.at[pl.ds(0, CH)], src_v[b], esem[b]).wait()
        pltpu.make_async_copy(dst_hbm.at[pl.ds(0, CH)], dst_v[b], esem[b]).wait()
        pltpu.make_async_copy(rt_hbm.at[pl.ds(0, CH)], rt_v[b], esem[b]).wait()

    def do_x(b):
        for i in range(CH // 16):
            sl = pl.ds(16 * i, 16)
            sidx_v[b][sl] = src_v[b][sl] * NUM_REL + rt_v[b][sl]
            dsti_v[b][sl] = dst_v[b][sl]

    def fire_g(b):
        pltpu.async_copy(s3_hbm.at[sidx_v[b]], srow_v[b], gsem[b])
        pltpu.async_copy(p_hbm.at[src_v[b]], prow_v[b], gsem[b])

    def wait_g(b):
        pltpu.make_async_copy(s3_hbm.at[sidx_v[b]], srow_v[b], gsem[b]).wait()
        pltpu.make_async_copy(p_hbm.at[src_v[b]], prow_v[b], gsem[b]).wait()

    def do_c(b):
        @plsc.parallel_loop(0, CH, 1, unroll=8)
        def _scale(e):
            row = srow_v[b][e, pl.ds(0, 16)]
            tt = jnp.exp(jnp.maximum(row, 0.2 * row))
            rowx = jnp.where(expmask, tt, row)
            for h in range(HEADS):
                exs = rowx[h]
                s0 = pl.ds(h * 2 * 16, 16)
                s1 = pl.ds((h * 2 + 1) * 16, 16)
                msgc_v[e, s0] = prow_v[b][e, s0] * exs
                msgc_v[e, s1] = prow_v[b][e, s1] * exs
            msgc_v[e, pl.ds(HD, 16)] = rowx

    def fire_s(b):
        pltpu.async_copy(msgc_v, comb_sh.at[dsti_v[b]], msem, add=True)

    def wait_s(b):
        pltpu.make_async_copy(msgc_v, comb_sh.at[dsti_v[b]], msem).wait()

    def sub_a(b):
        wait_l(b)
        do_x(b)
        fire_g(b)

    def sub_b(c, b, first=False, last=False):
        wait_g(b)
        if not first:
            wait_s(1 - b)
        do_c(b)
        fire_s(b)
        if not last:
            if isinstance(c, int):
                if c + 2 < NCHUNK:
                    fire_l(c + 2, b)
            else:
                @pl.when(c + 2 < NCHUNK)
                def _():
                    fire_l(c + 2, b)

    fire_l(0, 0)
    fire_l(1, 1)
    sub_a(0)
    sub_b(0, 0, first=True)
    sub_a(1)
    sub_b(1, 1)

    def _pipe(k, _):
        c = 2 * k + 2
        sub_a(0)
        sub_b(c, 0)
        sub_a(1)
        sub_b(c + 1, 1)
        return 0

    lax.fori_loop(0, (NCHUNK - 3) // 2, _pipe, 0)

    sub_a(0)
    sub_b(NCHUNK - 1, 0, last=True)
    wait_s(0)

    plsc.subcore_barrier()

    r0 = sid * RPT
    pltpu.sync_copy(comb_sh.at[pl.ds(r0, RPT)], accs_hbm.at[cid, pl.ds(r0, RPT)])

    @pl.when(sid == 0)
    def _wtail():
        pltpu.sync_copy(comb_sh.at[pl.ds(NS * RPT, TAIL)],
                        accs_hbm.at[cid, pl.ds(NS * RPT, TAIL)])


def _k2(s3r, p, src, dst, rt):
    mesh = plsc.VectorSubcoreMesh(core_axis_name="c", subcore_axis_name="s")
    ivec = pltpu.VMEM((CH,), jnp.int32)
    f = functools.partial(
        pl.kernel,
        out_type=jax.ShapeDtypeStruct((NC, N, CW), jnp.float32),
        mesh=mesh,
        compiler_params=pltpu.CompilerParams(needs_layout_passes=False,
                                             use_tc_tiling_on_sc=False),
        scratch_types=[
            pltpu.VMEM_SHARED((N, CW), jnp.float32),
            [ivec, ivec], [ivec, ivec], [ivec, ivec], [ivec, ivec],
            [ivec, ivec],
            [pltpu.VMEM((CH, SW), jnp.float32), pltpu.VMEM((CH, SW), jnp.float32)],
            [pltpu.VMEM((CH, HD), jnp.float32), pltpu.VMEM((CH, HD), jnp.float32)],
            pltpu.VMEM((CH, CW), jnp.float32),
            [pltpu.SemaphoreType.DMA, pltpu.SemaphoreType.DMA],
            [pltpu.SemaphoreType.DMA, pltpu.SemaphoreType.DMA],
            pltpu.SemaphoreType.DMA,
        ],
    )(_k2_body)
    return f(s3r, p, src, dst, rt)


def _k3_body(acc_ref, d_ref, out_ref):
    a = acc_ref[0] + acc_ref[1]
    d = d_ref[0] + d_ref[1]
    bias = d[:, 4:5]
    parts = []
    for h in range(HEADS):
        den = d[:, h:h + 1] + EPS
        parts.append(a[:, h * OUT_DIM:(h + 1) * OUT_DIM] / den + bias)
    out_ref[...] = jnp.concatenate(parts, axis=1)


def _k3(acc, dacc):
    blk = 1000
    grid = N // blk
    return pl.pallas_call(
        _k3_body,
        grid=(grid,),
        in_specs=[
            pl.BlockSpec((NC, blk, HD), lambda i: (0, i, 0)),
            pl.BlockSpec((NC, blk, SW), lambda i: (0, i, 0)),
        ],
        out_specs=pl.BlockSpec((blk, HD), lambda i: (i, 0)),
        out_shape=jax.ShapeDtypeStruct((N, HD), jnp.float32),
    )(acc, dacc)


def kernel(node_emb, edge_index, edge_type, W, attn_vec, rel_bias):
    wcat = W.reshape(HD, IN_DIM)
    tmp = attn_vec.transpose(1, 0, 2)
    a2 = jnp.zeros((NUM_REL, SW, HEADS, OUT_DIM), jnp.float32)
    a2 = a2.at[:, jnp.arange(HEADS), jnp.arange(HEADS), :].set(tmp)
    a2 = a2.reshape(NUM_REL * SW, HD)
    brow = jnp.zeros((NUM_REL, SW), jnp.float32).at[:, 4].set(rel_bias)
    brow = brow.reshape(1, NUM_REL * SW)

    p, s3 = _k1(node_emb, wcat, a2, brow)
    s3r = s3.reshape(N * NUM_REL, SW)

    src = edge_index[0].astype(jnp.int32)
    dst = edge_index[1].astype(jnp.int32)
    rt = edge_type.astype(jnp.int32)

    combs = _k2(s3r, p, src, dst, rt)
    return _k3(combs[:, :, :HD], combs[:, :, HD:])


# speedup vs baseline: 3.0714x; 1.3666x over previous
import functools

import jax
import jax.numpy as jnp
from jax import lax
from jax.experimental import pallas as pl
from jax.experimental.pallas import tpu as pltpu
from jax.experimental.pallas import tpu_sc as plsc

N = 10000
E = 320000
IN_DIM = 128
OUT_DIM = 32
HEADS = 4
NUM_REL = 48
EPS = 1e-16
HD = HEADS * OUT_DIM
SW = 16
CW = HD + SW
NC = 2
NS = 16
NW = NC * NS
EPW = E // NW
CH = 80
NCHUNK = EPW // CH
RPT = 624
TAIL = N - NS * RPT


def _k1_body(x_ref, wcat_ref, a2_ref, brow_ref, p_ref, s3_ref):
    x = x_ref[...]
    p = lax.dot_general(x, wcat_ref[...], (((1,), (1,)), ((), ())),
                        preferred_element_type=jnp.float32)
    p_ref[...] = p
    s3 = lax.dot_general(p, a2_ref[...], (((1,), (1,)), ((), ())),
                         preferred_element_type=jnp.float32)
    s3_ref[...] = s3 + brow_ref[...]


def _k1(x, wcat, a2, brow):
    blk = 1000
    grid = N // blk
    return pl.pallas_call(
        _k1_body,
        grid=(grid,),
        in_specs=[
            pl.BlockSpec((blk, IN_DIM), lambda i: (i, 0)),
            pl.BlockSpec((HD, IN_DIM), lambda i: (0, 0)),
            pl.BlockSpec((NUM_REL * SW, HD), lambda i: (0, 0)),
            pl.BlockSpec((1, NUM_REL * SW), lambda i: (0, 0)),
        ],
        out_specs=[
            pl.BlockSpec((blk, HD), lambda i: (i, 0)),
            pl.BlockSpec((blk, NUM_REL * SW), lambda i: (i, 0)),
        ],
        out_shape=[
            jax.ShapeDtypeStruct((N, HD), jnp.float32),
            jax.ShapeDtypeStruct((N, NUM_REL * SW), jnp.float32),
        ],
    )(x, wcat, a2, brow)


def _k2_body(s3_hbm, p_hbm, src_hbm, dst_hbm, rt_hbm,
             accs_hbm, daccs_hbm,
             acc_sh, dacc_sh,
             src_v, dst_v, rt_v, sidx_v, dsti_v, srow_v, prow_v,
             esem, gsem, ssem):
    cid = lax.axis_index("c")
    sid = lax.axis_index("s")
    wid = sid * NC + cid

    @plsc.parallel_loop(0, CH, 1, unroll=8)
    def _zrow(i):
        for k in range(HD // 16):
            prow_v[0][i, pl.ds(16 * k, 16)] = jnp.zeros((16,), jnp.float32)
        srow_v[0][i, pl.ds(0, 16)] = jnp.zeros((16,), jnp.float32)

    for j in range(7):
        pltpu.sync_copy(prow_v[0], acc_sh.at[pl.ds(sid * RPT + j * CH, CH)])
        pltpu.sync_copy(srow_v[0], dacc_sh.at[pl.ds(sid * RPT + j * CH, CH)])
    pltpu.sync_copy(prow_v[0].at[pl.ds(0, 64)], acc_sh.at[pl.ds(sid * RPT + 7 * CH, 64)])
    pltpu.sync_copy(srow_v[0].at[pl.ds(0, 64)], dacc_sh.at[pl.ds(sid * RPT + 7 * CH, 64)])

    @pl.when(sid == 0)
    def _ztail():
        pltpu.sync_copy(prow_v[0].at[pl.ds(0, TAIL)], acc_sh.at[pl.ds(NS * RPT, TAIL)])
        pltpu.sync_copy(srow_v[0].at[pl.ds(0, TAIL)], dacc_sh.at[pl.ds(NS * RPT, TAIL)])

    plsc.subcore_barrier()

    lanes = lax.iota(jnp.int32, 16)
    expmask = lanes < HEADS

    def fire_l(c, b):
        base = wid * EPW + c * CH
        pltpu.async_copy(src_hbm.at[pl.ds(base, CH)], src_v[b], esem[b])
        pltpu.async_copy(dst_hbm.at[pl.ds(base, CH)], dst_v[b], esem[b])
        pltpu.async_copy(rt_hbm.at[pl.ds(base, CH)], rt_v[b], esem[b])

    def wait_l(b):
        pltpu.make_async_copy(src_hbm.at[pl.ds(0, CH)], src_v[b], esem[b]).wait()
        pltpu.make_async_copy(dst_hbm.at[pl.ds(0, CH)], dst_v[b], esem[b]).wait()
        pltpu.make_async_copy(rt_hbm.at[pl.ds(0, CH)], rt_v[b], esem[b]).wait()

    def do_x(b):
        for i in range(CH // 16):
            sl = pl.ds(16 * i, 16)
            sidx_v[b][sl] = src_v[b][sl] * NUM_REL + rt_v[b][sl]
            dsti_v[b][sl] = dst_v[b][sl]

    def fire_g(b):
        pltpu.async_copy(s3_hbm.at[sidx_v[b]], srow_v[b], gsem[b])
        pltpu.async_copy(p_hbm.at[src_v[b]], prow_v[b], gsem[b])

    def wait_g(b):
        pltpu.make_async_copy(s3_hbm.at[sidx_v[b]], srow_v[b], gsem[b]).wait()
        pltpu.make_async_copy(p_hbm.at[src_v[b]], prow_v[b], gsem[b]).wait()

    def do_c(b):
        @plsc.parallel_loop(0, CH, 1, unroll=8)
        def _scale(e):
            row = srow_v[b][e, pl.ds(0, 16)]
            tt = jnp.exp(jnp.maximum(row, 0.2 * row))
            rowx = jnp.where(expmask, tt, row)
            for h in range(HEADS):
                exs = rowx[h]
                s0 = pl.ds(h * 2 * 16, 16)
                s1 = pl.ds((h * 2 + 1) * 16, 16)
                prow_v[b][e, s0] = prow_v[b][e, s0] * exs
                prow_v[b][e, s1] = prow_v[b][e, s1] * exs
            srow_v[b][e, pl.ds(0, 16)] = rowx

    def fire_s(b):
        pltpu.async_copy(srow_v[b], dacc_sh.at[dsti_v[b]], ssem[b], add=True)
        pltpu.async_copy(prow_v[b], acc_sh.at[dsti_v[b]], ssem[b], add=True)

    def wait_s(b):
        pltpu.make_async_copy(srow_v[b], dacc_sh.at[dsti_v[b]], ssem[b]).wait()
        pltpu.make_async_copy(prow_v[b], acc_sh.at[dsti_v[b]], ssem[b]).wait()

    def sub_a(b, first=False):
        wait_l(b)
        if not first:
            wait_s(b)
        do_x(b)
        fire_g(b)

    def sub_b(c, b, last=False):
        wait_g(b)
        do_c(b)
        fire_s(b)
        if not last:
            if isinstance(c, int):
                if c + 2 < NCHUNK:
                    fire_l(c + 2, b)
            else:
                @pl.when(c + 2 < NCHUNK)
                def _():
                    fire_l(c + 2, b)

    fire_l(0, 0)
    fire_l(1, 1)
    sub_a(0, first=True)
    sub_a(1, first=True)

    def _pipe(k, _):
        c = 2 * k
        sub_b(c, 0)
        sub_a(0)
        sub_b(c + 1, 1)

        @pl.when(c + 3 < NCHUNK)
        def _():
            sub_a(1)
        return 0

    lax.fori_loop(0, (NCHUNK - 1) // 2, _pipe, 0)

    sub_b(NCHUNK - 1, 0, last=True)
    wait_s(0)
    wait_s(1)

    plsc.subcore_barrier()

    r0 = sid * RPT
    pltpu.sync_copy(acc_sh.at[pl.ds(r0, RPT)], accs_hbm.at[cid, pl.ds(r0, RPT)])
    pltpu.sync_copy(dacc_sh.at[pl.ds(r0, RPT)], daccs_hbm.at[cid, pl.ds(r0, RPT)])

    @pl.when(sid == 0)
    def _wtail():
        pltpu.sync_copy(acc_sh.at[pl.ds(NS * RPT, TAIL)],
                        accs_hbm.at[cid, pl.ds(NS * RPT, TAIL)])
        pltpu.sync_copy(dacc_sh.at[pl.ds(NS * RPT, TAIL)],
                        daccs_hbm.at[cid, pl.ds(NS * RPT, TAIL)])


def _k2(s3r, p, src, dst, rt):
    mesh = plsc.VectorSubcoreMesh(core_axis_name="c", subcore_axis_name="s")
    ivec = pltpu.VMEM((CH,), jnp.int32)
    f = functools.partial(
        pl.kernel,
        out_type=[
            jax.ShapeDtypeStruct((NC, N, HD), jnp.float32),
            jax.ShapeDtypeStruct((NC, N, SW), jnp.float32),
        ],
        mesh=mesh,
        compiler_params=pltpu.CompilerParams(needs_layout_passes=False,
                                             use_tc_tiling_on_sc=False),
        scratch_types=[
            pltpu.VMEM_SHARED((N, HD), jnp.float32),
            pltpu.VMEM_SHARED((N, SW), jnp.float32),
            [ivec, ivec], [ivec, ivec], [ivec, ivec], [ivec, ivec],
            [ivec, ivec],
            [pltpu.VMEM((CH, SW), jnp.float32), pltpu.VMEM((CH, SW), jnp.float32)],
            [pltpu.VMEM((CH, HD), jnp.float32), pltpu.VMEM((CH, HD), jnp.float32)],
            [pltpu.SemaphoreType.DMA, pltpu.SemaphoreType.DMA],
            [pltpu.SemaphoreType.DMA, pltpu.SemaphoreType.DMA],
            [pltpu.SemaphoreType.DMA, pltpu.SemaphoreType.DMA],
        ],
    )(_k2_body)
    return f(s3r, p, src, dst, rt)


def _k3_body(acc_ref, d_ref, out_ref):
    a = acc_ref[0] + acc_ref[1]
    d = d_ref[0] + d_ref[1]
    bias = d[:, 4:5]
    parts = []
    for h in range(HEADS):
        den = d[:, h:h + 1] + EPS
        parts.append(a[:, h * OUT_DIM:(h + 1) * OUT_DIM] / den + bias)
    out_ref[...] = jnp.concatenate(parts, axis=1)


def _k3(acc, dacc):
    blk = 1000
    grid = N // blk
    return pl.pallas_call(
        _k3_body,
        grid=(grid,),
        in_specs=[
            pl.BlockSpec((NC, blk, HD), lambda i: (0, i, 0)),
            pl.BlockSpec((NC, blk, SW), lambda i: (0, i, 0)),
        ],
        out_specs=pl.BlockSpec((blk, HD), lambda i: (i, 0)),
        out_shape=jax.ShapeDtypeStruct((N, HD), jnp.float32),
    )(acc, dacc)


def kernel(node_emb, edge_index, edge_type, W, attn_vec, rel_bias):
    wcat = W.reshape(HD, IN_DIM)
    tmp = attn_vec.transpose(1, 0, 2)
    a2 = jnp.zeros((NUM_REL, SW, HEADS, OUT_DIM), jnp.float32)
    a2 = a2.at[:, jnp.arange(HEADS), jnp.arange(HEADS), :].set(tmp)
    a2 = a2.reshape(NUM_REL * SW, HD)
    brow = jnp.zeros((NUM_REL, SW), jnp.float32).at[:, 4].set(rel_bias)
    brow = brow.reshape(1, NUM_REL * SW)

    p, s3 = _k1(node_emb, wcat, a2, brow)
    s3r = s3.reshape(N * NUM_REL, SW)

    src = edge_index[0].astype(jnp.int32)
    dst = edge_index[1].astype(jnp.int32)
    rt = edge_type.astype(jnp.int32)

    accs, daccs = _k2(s3r, p, src, dst, rt)
    return _k3(accs, daccs)
